# baseline (speedup 1.0000x reference)
import functools

import jax
import jax.numpy as jnp
from jax import lax
from jax.experimental import pallas as pl
from jax.experimental.pallas import tpu as pltpu
from jax.experimental.pallas import tpu_sc as plsc

IN, HID, OUT = 784, 400, 10
THRESH = 0.5
NLANE = 16
NGROUP = HID // NLANE
NBLK1 = IN // NLANE
PAD1 = IN
PAD2 = HID
W1COLS = IN + 17
W2COLS = HID + 17
NEG = float("-inf")
POS = float("inf")


def _compact(src_load, nblocks, idx_ref, pad_idx):
    padv = jnp.full((NLANE,), pad_idx, jnp.int32)

    def fill(k, _):
        idx_ref[pl.ds(k * NLANE, NLANE)] = padv
        return 0

    lax.fori_loop(0, nblocks + 1, fill, 0)
    lanes = lax.iota(jnp.int32, NLANE)

    def blk(b, cntv):
        av = src_load(b)
        mask = av > 0.0
        x = jnp.where(mask, 1, 0).astype(jnp.int32)
        for kk in (1, 2, 4, 8):
            g = x.at[jnp.maximum(lanes - kk, 0)].get(
                mode="promise_in_bounds")
            x = x + jnp.where(lanes >= kk, g, 0)
        pos = cntv + x - 1
        plsc.store_scatter(idx_ref, [pos], lanes + b * NLANE, mask=mask)
        return cntv + jnp.full((NLANE,), x[NLANE - 1], jnp.int32)

    cntv = lax.fori_loop(0, nblocks, blk, jnp.zeros((NLANE,), jnp.int32))
    return cntv[0]


def _snn_body(inv_hbm, w1_hbm, w2_hbm, out_hbm,
              w1t, inv, w2t, statloc, statall, outbuf, idx1, idx2,
              sem, statsh):
    w = lax.axis_index("s")
    lanes = lax.iota(jnp.int32, NLANE)
    zero = jnp.zeros((NLANE,), jnp.float32)
    ninf = jnp.full((NLANE,), NEG)
    pinf = jnp.full((NLANE,), POS)

    @pl.when(w * 2 * NLANE < HID)
    def _stage1():
        for gl in range(2):
            g = 2 * w + gl

            @pl.when(g * NLANE < HID)
            def _():
                pltpu.async_copy(
                    w1_hbm.at[pl.ds(g * NLANE, NLANE), :],
                    w1t.at[pl.ds(gl * NLANE, NLANE), pl.ds(0, IN)], sem)

        @pl.when(w == 0)
        def _():
            pltpu.async_copy(w2_hbm, w2t.at[pl.ds(0, OUT), pl.ds(0, HID)],
                             sem)

        with jax.named_scope("p_inv_compact"):
            pltpu.sync_copy(inv_hbm, inv)
            cnt = _compact(lambda b: inv[pl.ds(b * NLANE, NLANE)],
                           NBLK1, idx1, PAD1)

        def wait_w1():
            pltpu.make_async_copy(
                w1_hbm.at[pl.ds(0, NLANE), :],
                w1t.at[pl.ds(0, NLANE), pl.ds(0, IN)], sem).wait()

        with jax.named_scope("p_w1_wait"):
            wait_w1()

            @pl.when((2 * w + 1) * NLANE < HID)
            def _():
                wait_w1()

        rows0 = lanes
        rows1 = lanes + NLANE
        for gl in range(2):
            plsc.store_scatter(
                w1t, [lanes + gl * NLANE,
                      jnp.full((NLANE,), PAD1, jnp.int32)], zero)

        nb = lax.shift_right_logical(cnt + 7, 3)

        def step(k, carry):
            s0, M0, m0, s1, M1, m1 = carry
            idxv = idx1[pl.ds(k * 8, NLANE)]
            for j in range(8):
                ci = jnp.full((NLANE,), idxv[j], jnp.int32)
                c0 = plsc.load_gather(w1t, [rows0, ci])
                c1 = plsc.load_gather(w1t, [rows1, ci])
                s0 = s0 + c0
                s1 = s1 + c1
                M0 = jnp.maximum(M0, s0)
                m0 = jnp.minimum(m0, s0)
                M1 = jnp.maximum(M1, s1)
                m1 = jnp.minimum(m1, s1)
            return (s0, M0, m0, s1, M1, m1)

        with jax.named_scope("p_loop1"):
            s0, M0, m0, s1, M1, m1 = lax.fori_loop(
                0, nb, step, (zero, ninf, pinf, zero, ninf, pinf))

        for gl, (s, M, m) in enumerate(((s0, M0, m0), (s1, M1, m1))):
            g = 2 * w + gl

            @pl.when(g * NLANE < HID)
            def _():
                statloc[0] = jnp.where(s > THRESH, 1.0, 0.0).astype(jnp.float32)
                statloc[1] = M
                statloc[2] = m
                pltpu.sync_copy(statloc, statsh.at[g])

    with jax.named_scope("p_barrier"):
        plsc.subcore_barrier()

    @pl.when(w == 0)
    def _stage2():
        pltpu.sync_copy(statsh, statall)

        def red(r, carry):
            Ma, ma = carry
            return (jnp.maximum(Ma, statall[r, 1]),
                    jnp.minimum(ma, statall[r, 2]))

        Ma, ma = lax.fori_loop(0, NGROUP, red, (ninf, pinf))
        smax1 = jnp.max(Ma)
        smin1 = jnp.min(ma)

        cnt2 = _compact(lambda r: statall[r, 0], NGROUP, idx2, PAD2)
        rows2 = jnp.minimum(lanes, OUT - 1)
        plsc.store_scatter(
            w2t, [lanes, jnp.full((NLANE,), PAD2, jnp.int32)], zero)
        pltpu.make_async_copy(
            w2_hbm, w2t.at[pl.ds(0, OUT), pl.ds(0, HID)], sem).wait()
        nb2 = lax.shift_right_logical(cnt2 + 7, 3)

        def step2(k, carry):
            s2, M2, m2 = carry
            idxv = idx2[pl.ds(k * 8, NLANE)]
            for j in range(8):
                ci = jnp.full((NLANE,), idxv[j], jnp.int32)
                col = plsc.load_gather(w2t, [rows2, ci])
                s2 = s2 + col
                M2 = jnp.maximum(M2, s2)
                m2 = jnp.minimum(m2, s2)
            return (s2, M2, m2)

        with jax.named_scope("p_loop2"):
            s2, M2, m2 = lax.fori_loop(
                0, nb2, step2,
                (zero, jnp.full((NLANE,), smax1), jnp.full((NLANE,), smin1)))

        valid = lanes < OUT
        smax = jnp.max(jnp.where(valid, M2, ninf))
        smin = jnp.min(jnp.where(valid, m2, pinf))
        outbuf[0] = jnp.where(s2 > THRESH, 1.0, 0.0).astype(jnp.float32)
        outbuf[1] = jnp.full((NLANE,), smax)
        outbuf[2] = jnp.full((NLANE,), smin)
        pltpu.sync_copy(outbuf, out_hbm)


@jax.jit
def _snn(input_vec, W1, W2):
    run = pl.kernel(
        _snn_body,
        out_type=jax.ShapeDtypeStruct((3, NLANE), jnp.float32),
        mesh=plsc.VectorSubcoreMesh(
            core_axis_name="c", subcore_axis_name="s", num_cores=1),
        compiler_params=pltpu.CompilerParams(
            use_tc_tiling_on_sc=False, needs_layout_passes=False),
        scratch_types=[
            pltpu.VMEM((2 * NLANE, W1COLS), jnp.float32),
            pltpu.VMEM((IN,), jnp.float32),
            pltpu.VMEM((NLANE, W2COLS), jnp.float32),
            pltpu.VMEM((3, NLANE), jnp.float32),
            pltpu.VMEM((2 * NGROUP, 3, NLANE), jnp.float32),
            pltpu.VMEM((3, NLANE), jnp.float32),
            pltpu.VMEM((IN + NLANE,), jnp.int32),
            pltpu.VMEM((HID + NLANE,), jnp.int32),
            pltpu.SemaphoreType.DMA,
            pltpu.VMEM_SHARED((2 * NGROUP, 3, NLANE), jnp.float32),
        ],
    )
    return run(input_vec, W1, W2)


def kernel(input_vec, W1, W2):
    out = _snn(input_vec, W1, W2)
    return out[0, :OUT], out[1, 0], out[2, 0]


# speedup vs baseline: 1.0704x; 1.0704x over previous
import functools

import jax
import jax.numpy as jnp
from jax import lax
from jax.experimental import pallas as pl
from jax.experimental.pallas import tpu as pltpu
from jax.experimental.pallas import tpu_sc as plsc

IN, HID, OUT = 784, 400, 10
THRESH = 0.5
NLANE = 16
NGROUP = HID // NLANE
NBLK1 = IN // NLANE
PAD1 = IN
PAD2 = HID
W1COLS = IN + 17
W2COLS = HID + 17
NEG = float("-inf")
POS = float("inf")


def _compact(src_load, nblocks, idx_ref, pad_idx):
    lanes = lax.iota(jnp.int32, NLANE)

    def blk(b, cntv):
        av = src_load(b)
        mask = av > 0.0
        cum = plsc.cumsum(jnp.where(mask, 1, 0).astype(jnp.int32))
        pos = cntv + cum - 1
        plsc.store_scatter(idx_ref, [pos], lanes + b * NLANE, mask=mask)
        return cntv + plsc.all_reduce_population_count(mask)

    cntv = lax.fori_loop(0, nblocks, blk, jnp.zeros((NLANE,), jnp.int32))
    plsc.store_scatter(idx_ref, [cntv + lanes],
                       jnp.full((NLANE,), pad_idx, jnp.int32))
    return cntv[0]


def _snn_body(inv_hbm, w1_hbm, w2_hbm, out_hbm,
              w1t, inv, w2t, statloc, statall, outbuf, idx1, idx2,
              sem, statsh):
    w = lax.axis_index("s")
    lanes = lax.iota(jnp.int32, NLANE)
    zero = jnp.zeros((NLANE,), jnp.float32)
    ninf = jnp.full((NLANE,), NEG)
    pinf = jnp.full((NLANE,), POS)

    @pl.when(w * 2 * NLANE < HID)
    def _stage1():
        for gl in range(2):
            g = 2 * w + gl

            @pl.when(g * NLANE < HID)
            def _():
                pltpu.async_copy(
                    w1_hbm.at[pl.ds(g * NLANE, NLANE), :],
                    w1t.at[pl.ds(gl * NLANE, NLANE), pl.ds(0, IN)], sem)

        @pl.when(w == 0)
        def _():
            pltpu.async_copy(w2_hbm, w2t.at[pl.ds(0, OUT), pl.ds(0, HID)],
                             sem)

        with jax.named_scope("p_inv_compact"):
            pltpu.sync_copy(inv_hbm, inv)
            cnt = _compact(lambda b: inv[pl.ds(b * NLANE, NLANE)],
                           NBLK1, idx1, PAD1)

        def wait_w1():
            pltpu.make_async_copy(
                w1_hbm.at[pl.ds(0, NLANE), :],
                w1t.at[pl.ds(0, NLANE), pl.ds(0, IN)], sem).wait()

        with jax.named_scope("p_w1_wait"):
            wait_w1()

            @pl.when((2 * w + 1) * NLANE < HID)
            def _():
                wait_w1()

        rows0 = lanes
        rows1 = lanes + NLANE
        for gl in range(2):
            plsc.store_scatter(
                w1t, [lanes + gl * NLANE,
                      jnp.full((NLANE,), PAD1, jnp.int32)], zero)

        nb = lax.shift_right_logical(cnt + 7, 3)

        def step(k, carry):
            s0, M0, m0, s1, M1, m1 = carry
            idxv = idx1[pl.ds(k * 8, NLANE)]
            for j in range(8):
                ci = jnp.full((NLANE,), idxv[j], jnp.int32)
                c0 = plsc.load_gather(w1t, [rows0, ci])
                c1 = plsc.load_gather(w1t, [rows1, ci])
                s0 = s0 + c0
                s1 = s1 + c1
                M0 = jnp.maximum(M0, s0)
                m0 = jnp.minimum(m0, s0)
                M1 = jnp.maximum(M1, s1)
                m1 = jnp.minimum(m1, s1)
            return (s0, M0, m0, s1, M1, m1)

        with jax.named_scope("p_loop1"):
            s0, M0, m0, s1, M1, m1 = lax.fori_loop(
                0, nb, step, (zero, ninf, pinf, zero, ninf, pinf))

        for gl, (s, M, m) in enumerate(((s0, M0, m0), (s1, M1, m1))):
            g = 2 * w + gl

            @pl.when(g * NLANE < HID)
            def _():
                statloc[0] = jnp.where(s > THRESH, 1.0, 0.0).astype(jnp.float32)
                statloc[1] = M
                statloc[2] = m
                pltpu.sync_copy(statloc, statsh.at[g])

    with jax.named_scope("p_barrier"):
        plsc.subcore_barrier()

    @pl.when(w == 0)
    def _stage2():
        pltpu.sync_copy(statsh, statall)

        def red(r, carry):
            Ma, ma = carry
            return (jnp.maximum(Ma, statall[r, 1]),
                    jnp.minimum(ma, statall[r, 2]))

        Ma, ma = lax.fori_loop(0, NGROUP, red, (ninf, pinf))
        smax1 = jnp.max(Ma)
        smin1 = jnp.min(ma)

        cnt2 = _compact(lambda r: statall[r, 0], NGROUP, idx2, PAD2)
        rows2 = jnp.minimum(lanes, OUT - 1)
        plsc.store_scatter(
            w2t, [lanes, jnp.full((NLANE,), PAD2, jnp.int32)], zero)
        pltpu.make_async_copy(
            w2_hbm, w2t.at[pl.ds(0, OUT), pl.ds(0, HID)], sem).wait()
        nb2 = lax.shift_right_logical(cnt2 + 7, 3)

        def step2(k, carry):
            s2, M2, m2 = carry
            idxv = idx2[pl.ds(k * 8, NLANE)]
            for j in range(8):
                ci = jnp.full((NLANE,), idxv[j], jnp.int32)
                col = plsc.load_gather(w2t, [rows2, ci])
                s2 = s2 + col
                M2 = jnp.maximum(M2, s2)
                m2 = jnp.minimum(m2, s2)
            return (s2, M2, m2)

        with jax.named_scope("p_loop2"):
            s2, M2, m2 = lax.fori_loop(
                0, nb2, step2,
                (zero, jnp.full((NLANE,), smax1), jnp.full((NLANE,), smin1)))

        valid = lanes < OUT
        smax = jnp.max(jnp.where(valid, M2, ninf))
        smin = jnp.min(jnp.where(valid, m2, pinf))
        outbuf[pl.ds(0, NLANE)] = jnp.where(s2 > THRESH, 1.0,
                                            0.0).astype(jnp.float32)
        outbuf[pl.ds(NLANE, NLANE)] = jnp.full((NLANE,), smax)
        outbuf[pl.ds(2 * NLANE, NLANE)] = jnp.full((NLANE,), smin)
        pltpu.sync_copy(outbuf, out_hbm)


@jax.jit
def _snn(input_vec, W1, W2):
    run = pl.kernel(
        _snn_body,
        out_type=jax.ShapeDtypeStruct((3 * NLANE,), jnp.float32),
        mesh=plsc.VectorSubcoreMesh(
            core_axis_name="c", subcore_axis_name="s", num_cores=1),
        compiler_params=pltpu.CompilerParams(
            use_tc_tiling_on_sc=False, needs_layout_passes=False),
        scratch_types=[
            pltpu.VMEM((2 * NLANE, W1COLS), jnp.float32),
            pltpu.VMEM((IN,), jnp.float32),
            pltpu.VMEM((NLANE, W2COLS), jnp.float32),
            pltpu.VMEM((3, NLANE), jnp.float32),
            pltpu.VMEM((2 * NGROUP, 3, NLANE), jnp.float32),
            pltpu.VMEM((3 * NLANE,), jnp.float32),
            pltpu.VMEM((IN + NLANE,), jnp.int32),
            pltpu.VMEM((HID + NLANE,), jnp.int32),
            pltpu.SemaphoreType.DMA,
            pltpu.VMEM_SHARED((2 * NGROUP, 3, NLANE), jnp.float32),
        ],
    )
    return run(input_vec, W1, W2)


def kernel(input_vec, W1, W2):
    out = _snn(input_vec, W1, W2)
    return out[:OUT], out[NLANE], out[2 * NLANE]
